# baseline (speedup 1.0000x reference)
import jax
import jax.numpy as jnp
from jax.experimental import pallas as pl
from jax.experimental.pallas import tpu as pltpu

_GRID = 10
_G2 = _GRID * _GRID
_D = 128
_CH = 2048
_B = 16384
_N = _B // _CH


def _nt(a, b):
    return jax.lax.dot_general(a, b, (((1,), (1,)), ((), ())),
                               preferred_element_type=jnp.float32)


def _tt(w, act):
    return jax.lax.dot_general(w, act, (((0,), (0,)), ((), ())),
                               preferred_element_type=jnp.float32)


def _tt_bf(w, act):
    return jax.lax.dot_general(w.astype(jnp.bfloat16), act.astype(jnp.bfloat16),
                               (((0,), (0,)), ((), ())),
                               preferred_element_type=jnp.float32)


def _pipeline(x_hbm, flat_ref,
              We1_ref, be1_ref, We2_ref, be2_ref, We3_ref, be3_ref,
              Wd1_ref, bd1_ref, Wd2_ref, bd2_ref, Wd3_ref, bd3_ref,
              Wg1_ref, bg1_ref, Wg2_ref, bg2_ref,
              out_hbm, x_vmem, o_vmem, in_sem, out_sem):
    eps = 1e-12

    flat = flat_ref[...]
    w2 = jnp.sum(flat * flat, axis=1, keepdims=True)
    flatm2 = -2.0 * flat
    row = jax.lax.broadcasted_iota(jnp.int32, (_G2, 1), 0)
    A = jnp.concatenate([We1_ref[...].T, Wd3_ref[...]],
                        axis=0).astype(jnp.bfloat16)
    Wd3 = Wd3_ref[...]
    be1c = be1_ref[...].T
    be2c, be3c = be2_ref[...].T, be3_ref[...].T
    bd1c, bd2c = bd1_ref[...].T, bd2_ref[...].T
    bd3c = bd3_ref[...].T
    bg1c, bg2c = bg1_ref[...].T, bg2_ref[...].T

    def in_copy(i, slot):
        return pltpu.make_async_copy(
            x_hbm.at[pl.ds(i * _CH, _CH), :], x_vmem.at[slot], in_sem.at[slot])

    def out_copy(i, slot):
        return pltpu.make_async_copy(
            o_vmem.at[slot], out_hbm.at[pl.ds(i * _CH, _CH), :], out_sem.at[slot])

    def compute(x):
        s = w2 + _nt(flatm2, x)
        smin = jnp.min(s, axis=0, keepdims=True)
        idx = jnp.min(jnp.where(s <= smin, row, _G2), axis=0, keepdims=True)
        wi = (idx // _GRID).astype(jnp.float32) * 0.1
        wj = (idx % _GRID).astype(jnp.float32) * 0.1

        xb = x.astype(jnp.bfloat16)
        ones_row = jnp.ones((1, _D), dtype=jnp.bfloat16)
        x2 = jax.lax.dot_general(ones_row, xb * xb, (((1,), (1,)), ((), ())),
                                 preferred_element_type=jnp.float32)
        x_norm = jnp.sqrt(x2)

        P = jax.lax.dot_general(A, xb, (((1,), (1,)), ((), ())),
                                preferred_element_type=jnp.float32)
        h = jnp.tanh(P[0:64] + be1c)
        C = P[64:128]

        h = jnp.tanh(_tt_bf(We2_ref[...], h) + be2c)
        z_c = _tt_bf(We3_ref[...], h) + be3c
        h = jnp.tanh(_tt_bf(Wd1_ref[...], z_c) + bd1c)
        h = jnp.tanh(_tt_bf(Wd2_ref[...], h) + bd2c)
        x_hat = _tt_bf(Wd3, h) + bd3c

        xxh = jnp.sum(h * C, axis=0, keepdims=True) + _nt(bd3_ref[...], x)
        xh2 = jnp.sum(x_hat * x_hat, axis=0, keepdims=True)
        diff2 = jnp.maximum(x2 - 2.0 * xxh + xh2, 0.0)
        rec_e = jnp.sqrt(diff2) / (x_norm + eps)
        rec_c = xxh / (x_norm * jnp.sqrt(xh2) + eps)

        z = jnp.concatenate([z_c, rec_e, rec_c, wi, wj], axis=0)
        g = jnp.tanh(_tt(Wg1_ref[...], z) + bg1c)
        logits = _tt(Wg2_ref[...], g) + bg2c
        m = jnp.max(logits, axis=0, keepdims=True)
        e = jnp.exp(logits - m)
        gamma = e / jnp.sum(e, axis=0, keepdims=True)
        return gamma.T

    in_copy(0, 0).start()

    for i in range(_N):
        slot = i % 2
        if i + 1 < _N:
            in_copy(i + 1, 1 - slot).start()
        in_copy(i, slot).wait()
        gamma = compute(x_vmem[slot])
        if i >= 2:
            out_copy(i - 2, slot).wait()
        o_vmem[slot] = gamma
        out_copy(i, slot).start()

    out_copy(_N - 2, (_N - 2) % 2).wait()
    out_copy(_N - 1, (_N - 1) % 2).wait()


def kernel(input, som_weights, We1, be1, We2, be2, We3, be3,
           Wd1, bd1, Wd2, bd2, Wd3, bd3, Wg1, bg1, Wg2, bg2):
    flat = som_weights.reshape(_G2, _D)

    def vmem_spec(a):
        nd = a.ndim
        return pl.BlockSpec(a.shape, lambda: (0,) * nd)

    weights = (flat,
               We1, be1.reshape(1, -1), We2, be2.reshape(1, -1),
               We3, be3.reshape(1, -1),
               Wd1, bd1.reshape(1, -1), Wd2, bd2.reshape(1, -1),
               Wd3, bd3.reshape(1, -1),
               Wg1, bg1.reshape(1, -1), Wg2, bg2.reshape(1, -1))

    gamma = pl.pallas_call(
        _pipeline,
        in_specs=[pl.BlockSpec(memory_space=pl.MemorySpace.ANY)]
                 + [vmem_spec(w) for w in weights],
        out_specs=pl.BlockSpec(memory_space=pl.MemorySpace.ANY),
        out_shape=jax.ShapeDtypeStruct((_B, 4), jnp.float32),
        scratch_shapes=[
            pltpu.VMEM((2, _CH, _D), jnp.float32),
            pltpu.VMEM((2, _CH, 4), jnp.float32),
            pltpu.SemaphoreType.DMA((2,)),
            pltpu.SemaphoreType.DMA((2,)),
        ],
    )(input, *weights)
    return gamma


# speedup vs baseline: 1.0384x; 1.0384x over previous
import jax
import jax.numpy as jnp
from jax.experimental import pallas as pl

_GRID = 10
_G2 = _GRID * _GRID
_D = 128
_BB = 8192

_OFF_FLAT = 0
_OFF_A = 104
_OFF_WE2 = 232
_OFF_WE3 = 296
_OFF_WD1 = 328
_OFF_WD2 = 336
_OFF_BD3 = 368
_OFF_BE1 = 376
_OFF_BE2 = 384
_OFF_BE3 = 392
_OFF_BD1 = 400
_OFF_BD2 = 408
_OFF_BG1 = 416
_OFF_BG2 = 424
_OFF_WG1 = 432
_OFF_WG2 = 440
_PACK_ROWS = 456


def _nt(a, b):
    return jax.lax.dot_general(a, b, (((1,), (1,)), ((), ())),
                               preferred_element_type=jnp.float32)


def _tt(w, act):
    return jax.lax.dot_general(w, act, (((0,), (0,)), ((), ())),
                               preferred_element_type=jnp.float32)


def _tt_bf(w, act):
    return jax.lax.dot_general(w.astype(jnp.bfloat16), act.astype(jnp.bfloat16),
                               (((0,), (0,)), ((), ())),
                               preferred_element_type=jnp.float32)


def _fused(x_ref, wp_ref, out_ref):
    eps = 1e-12
    x = x_ref[...]
    Wp = wp_ref[...]

    flatP = Wp[_OFF_FLAT:_OFF_FLAT + 104]
    row = jax.lax.broadcasted_iota(jnp.int32, (104, 1), 0)
    w2 = jnp.sum(flatP * flatP, axis=1, keepdims=True)
    w2 = jnp.where(row < _G2, w2, jnp.float32(1e30))
    s = w2 - 2.0 * _nt(flatP, x)
    smin = jnp.min(s, axis=0, keepdims=True)
    idx = jnp.min(jnp.where(s <= smin, row, _G2), axis=0, keepdims=True)
    wi = (idx // _GRID).astype(jnp.float32) * 0.1
    wj = (idx % _GRID).astype(jnp.float32) * 0.1

    xb = x.astype(jnp.bfloat16)
    ones_row = jnp.ones((1, _D), dtype=jnp.bfloat16)
    x2 = jax.lax.dot_general(ones_row, xb * xb, (((1,), (1,)), ((), ())),
                             preferred_element_type=jnp.float32)
    x_norm = jnp.sqrt(x2)

    A = Wp[_OFF_A:_OFF_A + 128].astype(jnp.bfloat16)
    P = jax.lax.dot_general(A, xb, (((1,), (1,)), ((), ())),
                            preferred_element_type=jnp.float32)
    be1c = Wp[_OFF_BE1:_OFF_BE1 + 1, 0:64].T
    h = jnp.tanh(P[0:64] + be1c)
    C = P[64:128]

    We2s = Wp[_OFF_WE2:_OFF_WE2 + 64, 0:32]
    We3s = Wp[_OFF_WE3:_OFF_WE3 + 32, 0:4]
    Wd1s = Wp[_OFF_WD1:_OFF_WD1 + 4, 0:32]
    Wd2s = Wp[_OFF_WD2:_OFF_WD2 + 32, 0:64]
    Wd3s = Wp[_OFF_A + 64:_OFF_A + 128]
    be2c = Wp[_OFF_BE2:_OFF_BE2 + 1, 0:32].T
    be3c = Wp[_OFF_BE3:_OFF_BE3 + 1, 0:4].T
    bd1c = Wp[_OFF_BD1:_OFF_BD1 + 1, 0:32].T
    bd2c = Wp[_OFF_BD2:_OFF_BD2 + 1, 0:64].T
    bd3row = Wp[_OFF_BD3:_OFF_BD3 + 1]
    bd3c = bd3row.T

    h = jnp.tanh(_tt_bf(We2s, h) + be2c)
    z_c = _tt_bf(We3s, h) + be3c
    h = jnp.tanh(_tt_bf(Wd1s, z_c) + bd1c)
    h = jnp.tanh(_tt_bf(Wd2s, h) + bd2c)
    x_hat = _tt_bf(Wd3s, h) + bd3c

    xxh = jnp.sum(h * C, axis=0, keepdims=True) + _nt(bd3row, x)
    xh2 = jnp.sum(x_hat * x_hat, axis=0, keepdims=True)
    diff2 = jnp.maximum(x2 - 2.0 * xxh + xh2, 0.0)
    rec_e = jnp.sqrt(diff2) / (x_norm + eps)
    rec_c = xxh / (x_norm * jnp.sqrt(xh2) + eps)

    Wg1s = Wp[_OFF_WG1:_OFF_WG1 + 8, 0:10]
    Wg2s = Wp[_OFF_WG2:_OFF_WG2 + 10, 0:4]
    bg1c = Wp[_OFF_BG1:_OFF_BG1 + 1, 0:10].T
    bg2c = Wp[_OFF_BG2:_OFF_BG2 + 1, 0:4].T
    z = jnp.concatenate([z_c, rec_e, rec_c, wi, wj], axis=0)
    g = jnp.tanh(_tt(Wg1s, z) + bg1c)
    logits = _tt(Wg2s, g) + bg2c
    m = jnp.max(logits, axis=0, keepdims=True)
    e = jnp.exp(logits - m)
    gamma = e / jnp.sum(e, axis=0, keepdims=True)
    out_ref[...] = gamma.T


def kernel(input, som_weights, We1, be1, We2, be2, We3, be3,
           Wd1, bd1, Wd2, bd2, Wd3, bd3, Wg1, bg1, Wg2, bg2):
    B = input.shape[0]
    flat = som_weights.reshape(_G2, _D)

    def pad_rows(a, r):
        return jnp.pad(a, ((0, r - a.shape[0]), (0, _D - a.shape[1])))

    Wpack = jnp.concatenate([
        pad_rows(flat, 104),
        pad_rows(We1.T, 64),
        pad_rows(Wd3, 64),
        pad_rows(We2, 64),
        pad_rows(We3, 32),
        pad_rows(Wd1, 8),
        pad_rows(Wd2, 32),
        pad_rows(bd3.reshape(1, -1), 8),
        pad_rows(be1.reshape(1, -1), 8),
        pad_rows(be2.reshape(1, -1), 8),
        pad_rows(be3.reshape(1, -1), 8),
        pad_rows(bd1.reshape(1, -1), 8),
        pad_rows(bd2.reshape(1, -1), 8),
        pad_rows(bg1.reshape(1, -1), 8),
        pad_rows(bg2.reshape(1, -1), 8),
        pad_rows(Wg1, 8),
        pad_rows(Wg2, 16),
    ], axis=0)

    gamma = pl.pallas_call(
        _fused,
        grid=(B // _BB,),
        in_specs=[pl.BlockSpec((_BB, _D), lambda i: (i, 0)),
                  pl.BlockSpec((_PACK_ROWS, _D), lambda i: (0, 0))],
        out_specs=pl.BlockSpec((_BB, 4), lambda i: (i, 0)),
        out_shape=jax.ShapeDtypeStruct((B, 4), jnp.float32),
    )(input, Wpack)
    return gamma


# speedup vs baseline: 1.1785x; 1.1349x over previous
import jax
import jax.numpy as jnp
from jax.experimental import pallas as pl
from jax.experimental.pallas import tpu as pltpu

_GRID = 10
_G2 = _GRID * _GRID
_D = 128
_BB = 8192


def _nt(a, b):
    return jax.lax.dot_general(a, b, (((1,), (1,)), ((), ())),
                               preferred_element_type=jnp.float32)


def _tt(w, act):
    return jax.lax.dot_general(w, act, (((0,), (0,)), ((), ())),
                               preferred_element_type=jnp.float32)


def _tt_bf(w, act):
    return jax.lax.dot_general(w.astype(jnp.bfloat16), act.astype(jnp.bfloat16),
                               (((0,), (0,)), ((), ())),
                               preferred_element_type=jnp.float32)


def _fused(x_ref, flat_hbm,
           We1_hbm, be1_hbm, We2_hbm, be2_hbm, We3_hbm, be3_hbm,
           Wd1_hbm, bd1_hbm, Wd2_hbm, bd2_hbm, Wd3_hbm, bd3_hbm,
           Wg1_hbm, bg1_hbm, Wg2_hbm, bg2_hbm,
           out_ref,
           flat_v, We1_v, be1_v, We2_v, be2_v, We3_v, be3_v,
           Wd1_v, bd1_v, Wd2_v, bd2_v, Wd3_v, bd3_v,
           Wg1_v, bg1_v, Wg2_v, bg2_v, sems):
    eps = 1e-12
    i = pl.program_id(0)

    hbm = (flat_hbm, We1_hbm, be1_hbm, We2_hbm, be2_hbm, We3_hbm, be3_hbm,
           Wd1_hbm, bd1_hbm, Wd2_hbm, bd2_hbm, Wd3_hbm, bd3_hbm,
           Wg1_hbm, bg1_hbm, Wg2_hbm, bg2_hbm)
    vmem = (flat_v, We1_v, be1_v, We2_v, be2_v, We3_v, be3_v,
            Wd1_v, bd1_v, Wd2_v, bd2_v, Wd3_v, bd3_v,
            Wg1_v, bg1_v, Wg2_v, bg2_v)

    @pl.when(i == 0)
    def _():
        for k, (src, dst) in enumerate(zip(hbm, vmem)):
            pltpu.make_async_copy(src, dst, sems.at[k]).start()
        for k, (src, dst) in enumerate(zip(hbm, vmem)):
            pltpu.make_async_copy(src, dst, sems.at[k]).wait()

    x = x_ref[...]
    flat = flat_v[...]

    w2 = jnp.sum(flat * flat, axis=1, keepdims=True)
    s = w2 - 2.0 * _nt(flat, x)
    smin = jnp.min(s, axis=0, keepdims=True)
    row = jax.lax.broadcasted_iota(jnp.int32, (_G2, 1), 0)
    idx = jnp.min(jnp.where(s <= smin, row, _G2), axis=0, keepdims=True)
    wi = (idx // _GRID).astype(jnp.float32) * 0.1
    wj = (idx % _GRID).astype(jnp.float32) * 0.1

    xb = x.astype(jnp.bfloat16)
    ones_row = jnp.ones((1, _D), dtype=jnp.bfloat16)
    x2 = jax.lax.dot_general(ones_row, xb * xb, (((1,), (1,)), ((), ())),
                             preferred_element_type=jnp.float32)
    x_norm = jnp.sqrt(x2)

    A = jnp.concatenate([We1_v[...].T, Wd3_v[...]],
                        axis=0).astype(jnp.bfloat16)
    P = jax.lax.dot_general(A, xb, (((1,), (1,)), ((), ())),
                            preferred_element_type=jnp.float32)
    h = jnp.tanh(P[0:64] + be1_v[...].T)
    C = P[64:128]

    h = jnp.tanh(_tt_bf(We2_v[...], h) + be2_v[...].T)
    z_c = _tt_bf(We3_v[...], h) + be3_v[...].T
    h = jnp.tanh(_tt_bf(Wd1_v[...], z_c) + bd1_v[...].T)
    h = jnp.tanh(_tt_bf(Wd2_v[...], h) + bd2_v[...].T)
    x_hat = _tt_bf(Wd3_v[...], h) + bd3_v[...].T

    xxh = jnp.sum(h * C, axis=0, keepdims=True) + _nt(bd3_v[...], x)
    xh2 = jnp.sum(x_hat * x_hat, axis=0, keepdims=True)
    diff2 = jnp.maximum(x2 - 2.0 * xxh + xh2, 0.0)
    rec_e = jnp.sqrt(diff2) / (x_norm + eps)
    rec_c = xxh / (x_norm * jnp.sqrt(xh2) + eps)

    z = jnp.concatenate([z_c, rec_e, rec_c, wi, wj], axis=0)
    g = jnp.tanh(_tt(Wg1_v[...], z) + bg1_v[...].T)
    logits = _tt(Wg2_v[...], g) + bg2_v[...].T
    m = jnp.max(logits, axis=0, keepdims=True)
    e = jnp.exp(logits - m)
    gamma = e / jnp.sum(e, axis=0, keepdims=True)
    out_ref[...] = gamma.T


def kernel(input, som_weights, We1, be1, We2, be2, We3, be3,
           Wd1, bd1, Wd2, bd2, Wd3, bd3, Wg1, bg1, Wg2, bg2):
    B = input.shape[0]
    flat = som_weights.reshape(_G2, _D)

    weights = (flat,
               We1, be1.reshape(1, -1), We2, be2.reshape(1, -1),
               We3, be3.reshape(1, -1),
               Wd1, bd1.reshape(1, -1), Wd2, bd2.reshape(1, -1),
               Wd3, bd3.reshape(1, -1),
               Wg1, bg1.reshape(1, -1), Wg2, bg2.reshape(1, -1))

    gamma = pl.pallas_call(
        _fused,
        grid=(B // _BB,),
        in_specs=[pl.BlockSpec((_BB, _D), lambda i: (i, 0))]
                 + [pl.BlockSpec(memory_space=pl.MemorySpace.ANY)
                    for _ in weights],
        out_specs=pl.BlockSpec((_BB, 4), lambda i: (i, 0)),
        out_shape=jax.ShapeDtypeStruct((B, 4), jnp.float32),
        scratch_shapes=[pltpu.VMEM(w.shape, jnp.float32) for w in weights]
                       + [pltpu.SemaphoreType.DMA((len(weights),))],
    )(input, *weights)
    return gamma


# speedup vs baseline: 1.2405x; 1.0526x over previous
import jax
import jax.numpy as jnp
from jax.experimental import pallas as pl

_GRID = 10
_G2 = _GRID * _GRID
_D = 128
_BB = 8192


def _nt(a, b):
    return jax.lax.dot_general(a, b, (((1,), (1,)), ((), ())),
                               preferred_element_type=jnp.float32)


def _tt(w, act):
    return jax.lax.dot_general(w, act, (((0,), (0,)), ((), ())),
                               preferred_element_type=jnp.float32)


def _tt_bf(w, act):
    return jax.lax.dot_general(w.astype(jnp.bfloat16), act.astype(jnp.bfloat16),
                               (((0,), (0,)), ((), ())),
                               preferred_element_type=jnp.float32)


def _fused(x_ref, flat_ref,
           We1_ref, be1_ref, We2_ref, be2_ref, We3_ref, be3_ref,
           Wd1_ref, bd1_ref, Wd2_ref, bd2_ref, Wd3_ref, bd3_ref,
           Wg1_ref, bg1_ref, Wg2_ref, bg2_ref,
           out_ref):
    eps = 1e-12
    x = x_ref[...]
    flat = flat_ref[...]

    w2 = jnp.sum(flat * flat, axis=1, keepdims=True)
    s = w2 + _nt(-2.0 * flat, x)
    smin = jnp.min(s, axis=0, keepdims=True)
    row = jax.lax.broadcasted_iota(jnp.int32, (_G2, 1), 0)
    idx = jnp.min(jnp.where(s <= smin, row, _G2), axis=0, keepdims=True)
    wi = (idx // _GRID).astype(jnp.float32) * 0.1
    wj = (idx % _GRID).astype(jnp.float32) * 0.1

    ones_row = jnp.ones((1, _D), dtype=jnp.bfloat16)
    xb = x.astype(jnp.bfloat16)
    x2 = jax.lax.dot_general(ones_row, xb * xb, (((1,), (1,)), ((), ())),
                             preferred_element_type=jnp.float32)
    x_norm = jnp.sqrt(x2)

    A = jnp.concatenate([We1_ref[...].T, Wd3_ref[...]], axis=0)
    P = jax.lax.dot_general(A.astype(jnp.bfloat16),
                            x.astype(jnp.bfloat16),
                            (((1,), (1,)), ((), ())),
                            preferred_element_type=jnp.float32)
    h = jnp.tanh(P[0:64] + be1_ref[...].T)
    C = P[64:128]

    h = jnp.tanh(_tt_bf(We2_ref[...], h) + be2_ref[...].T)
    z_c = _tt_bf(We3_ref[...], h) + be3_ref[...].T
    h = jnp.tanh(_tt_bf(Wd1_ref[...], z_c) + bd1_ref[...].T)
    h = jnp.tanh(_tt_bf(Wd2_ref[...], h) + bd2_ref[...].T)
    x_hat = _tt_bf(Wd3_ref[...], h) + bd3_ref[...].T

    xxh = jnp.sum(h * C, axis=0, keepdims=True) + _nt(bd3_ref[...], x)
    xh2 = jnp.sum(x_hat * x_hat, axis=0, keepdims=True)
    diff2 = jnp.maximum(x2 - 2.0 * xxh + xh2, 0.0)
    rec_e = jnp.sqrt(diff2) / (x_norm + eps)
    rec_c = xxh / (x_norm * jnp.sqrt(xh2) + eps)

    z = jnp.concatenate([z_c, rec_e, rec_c, wi, wj], axis=0)
    g = jnp.tanh(_tt(Wg1_ref[...], z) + bg1_ref[...].T)
    logits = _tt(Wg2_ref[...], g) + bg2_ref[...].T
    m = jnp.max(logits, axis=0, keepdims=True)
    e = jnp.exp(logits - m)
    gamma = e / jnp.sum(e, axis=0, keepdims=True)
    out_ref[...] = gamma.T


def kernel(input, som_weights, We1, be1, We2, be2, We3, be3,
           Wd1, bd1, Wd2, bd2, Wd3, bd3, Wg1, bg1, Wg2, bg2):
    B = input.shape[0]
    flat = som_weights.reshape(_G2, _D)

    def full_spec(a):
        nd = a.ndim
        return pl.BlockSpec(a.shape, lambda i: (0,) * nd)

    weights = (flat,
               We1, be1.reshape(1, -1), We2, be2.reshape(1, -1),
               We3, be3.reshape(1, -1),
               Wd1, bd1.reshape(1, -1), Wd2, bd2.reshape(1, -1),
               Wd3, bd3.reshape(1, -1),
               Wg1, bg1.reshape(1, -1), Wg2, bg2.reshape(1, -1))

    gamma = pl.pallas_call(
        _fused,
        grid=(B // _BB,),
        in_specs=[pl.BlockSpec((_BB, _D), lambda i: (i, 0))]
                 + [full_spec(w) for w in weights],
        out_specs=pl.BlockSpec((_BB, 4), lambda i: (i, 0)),
        out_shape=jax.ShapeDtypeStruct((B, 4), jnp.float32),
    )(input, *weights)
    return gamma
